# baseline (speedup 1.0000x reference)
import functools

import jax
import jax.numpy as jnp
from jax import lax
from jax.experimental import pallas as pl
from jax.experimental.pallas import tpu as pltpu
from jax.experimental.pallas import tpu_sc as plsc

N = 10000
E = 320000
NUM_GRAPHS = 64
OUT_DIM = 10
EPS = 1e-7

NC = 2
NS = 16
D = 64
EPB = 128
E_PER_SUB = 10240
E_PAD = NC * NS * E_PER_SUB
E_PER_CORE = E_PAD // NC
BLOCKS = E_PER_SUB // EPB
N_PAD = 10112
ROWS_PER_SUB = N_PAD // NS
ZROWS = 128
JUNK_ROW = N_PAD - 1


def _edge_body(h_hbm, e_hbm, src_hbm, dst_hbm, out_hbm,
               srcb, dstb, hrows, erows, wbuf, mwbuf, zbuf,
               acc, s_src, s_dst, s_h, s_e, s_sw, s_sm):
    c = lax.axis_index("c")
    s = lax.axis_index("s")
    base = c * E_PER_CORE + s * E_PER_SUB

    zv = jnp.zeros((16,), jnp.float32)

    @pl.loop(0, ZROWS)
    def _(r):
        for g in range(D // 16):
            zbuf[r, pl.ds(g * 16, 16)] = zv

    for a in range(2):
        for k in range(ROWS_PER_SUB // ZROWS):
            pltpu.sync_copy(
                zbuf, acc.at[a, pl.ds(s * ROWS_PER_SUB + k * ZROWS, ZROWS)])
        rem = ROWS_PER_SUB % ZROWS
        if rem:
            pltpu.sync_copy(
                zbuf.at[pl.ds(0, rem)],
                acc.at[a, pl.ds(s * ROWS_PER_SUB
                                + (ROWS_PER_SUB // ZROWS) * ZROWS, rem)])
    plsc.subcore_barrier()

    @pl.loop(0, BLOCKS)
    def _(j):
        off = base + j * EPB
        cp_s = pltpu.async_copy(src_hbm.at[pl.ds(off, EPB)], srcb, s_src)
        cp_d = pltpu.async_copy(dst_hbm.at[pl.ds(off, EPB)], dstb, s_dst)
        cp_s.wait()
        cp_d.wait()
        cp_h = pltpu.async_copy(h_hbm.at[srcb], hrows, s_h)
        cp_e = pltpu.async_copy(e_hbm.at[pl.ds(off, EPB)], erows, s_e)
        cp_h.wait()
        cp_e.wait()

        @pl.loop(0, EPB)
        def _(r):
            for g in range(D // 16):
                sl = pl.ds(g * 16, 16)
                m = jnp.maximum(hrows[r, sl] + erows[r, sl], 0.0) + EPS
                w = jnp.exp(m)
                wbuf[r, sl] = w
                mwbuf[r, sl] = m * w

        cp_w = pltpu.async_copy(wbuf, acc.at[0].at[dstb], s_sw, add=True)
        cp_m = pltpu.async_copy(mwbuf, acc.at[1].at[dstb], s_sm, add=True)
        cp_w.wait()
        cp_m.wait()

    plsc.subcore_barrier()
    for a in range(2):
        pltpu.sync_copy(
            acc.at[a, pl.ds(s * ROWS_PER_SUB, ROWS_PER_SUB)],
            out_hbm.at[c, a, pl.ds(s * ROWS_PER_SUB, ROWS_PER_SUB)])


@jax.jit
def _edge_pass(h, e, src, dst):
    mesh = plsc.VectorSubcoreMesh(core_axis_name="c", subcore_axis_name="s")
    f = pl.kernel(
        _edge_body,
        out_type=jax.ShapeDtypeStruct((NC, 2, N_PAD, D), jnp.float32),
        mesh=mesh,
        scratch_types=[
            pltpu.VMEM((EPB,), jnp.int32),
            pltpu.VMEM((EPB,), jnp.int32),
            pltpu.VMEM((EPB, D), jnp.float32),
            pltpu.VMEM((EPB, D), jnp.float32),
            pltpu.VMEM((EPB, D), jnp.float32),
            pltpu.VMEM((EPB, D), jnp.float32),
            pltpu.VMEM((ZROWS, D), jnp.float32),
            pltpu.VMEM_SHARED((2, N_PAD, D), jnp.float32),
            pltpu.SemaphoreType.DMA,
            pltpu.SemaphoreType.DMA,
            pltpu.SemaphoreType.DMA,
            pltpu.SemaphoreType.DMA,
            pltpu.SemaphoreType.DMA,
            pltpu.SemaphoreType.DMA,
        ],
        compiler_params=pltpu.CompilerParams(use_tc_tiling_on_sc=False),
    )
    return f(h, e, src, dst)


def _lin_body(a_ref, w_ref, b_ref, o_ref):
    o_ref[...] = (
        jnp.dot(a_ref[...], w_ref[...], preferred_element_type=jnp.float32)
        + b_ref[...])


def _linear(a, w, b, block_rows=None):
    m, k = a.shape
    n = w.shape[1]
    if block_rows is None:
        return pl.pallas_call(
            _lin_body,
            out_shape=jax.ShapeDtypeStruct((m, n), jnp.float32),
        )(a, w, b.reshape(1, n))
    grid = m // block_rows
    return pl.pallas_call(
        _lin_body,
        grid=(grid,),
        in_specs=[
            pl.BlockSpec((block_rows, k), lambda i: (i, 0)),
            pl.BlockSpec((k, n), lambda i: (0, 0)),
            pl.BlockSpec((1, n), lambda i: (0, 0)),
        ],
        out_specs=pl.BlockSpec((block_rows, n), lambda i: (i, 0)),
        out_shape=jax.ShapeDtypeStruct((m, n), jnp.float32),
    )(a, w, b.reshape(1, n))


RB = 2000


def _combA_body(xd_ref, w1_ref, b1_ref, *rest):
    p_refs = rest[:-2]
    h2_ref, sums_ref = rest[-2:]
    aggs = []
    for pr in p_refs:
        w = pr[0, 0, :, :] + pr[1, 0, :, :]
        mw = pr[0, 1, :, :] + pr[1, 1, :, :]
        aggs.append(mw / (w + 1e-16))
    agg = aggs[0] if len(aggs) == 1 else jnp.concatenate(aggs, axis=1)
    out = agg + xd_ref[...]
    h2 = (jnp.dot(out, w1_ref[...], preferred_element_type=jnp.float32)
          + b1_ref[...])
    h2_ref[...] = h2
    s1 = jnp.sum(h2, axis=0, keepdims=True)
    s2 = jnp.sum(h2 * h2, axis=0, keepdims=True)

    @pl.when(pl.program_id(0) == 0)
    def _():
        sums_ref[...] = jnp.zeros_like(sums_ref)

    sums_ref[0:1, :] += s1
    sums_ref[1:2, :] += s2


def _combB_body(h2_ref, sums_ref, g_ref, bt_ref, w2_ref, b2_ref, o_ref):
    mu = sums_ref[0:1, :] / N
    var = sums_ref[1:2, :] / N - mu * mu
    h2 = (h2_ref[...] - mu) * lax.rsqrt(var + 1e-5) * g_ref[...] + bt_ref[...]
    h2 = jnp.maximum(h2, 0.0)
    z = (jnp.dot(h2, w2_ref[...], preferred_element_type=jnp.float32)
         + b2_ref[...])
    o_ref[...] = jnp.maximum(z, 0.0)


def _combine(parts, xd, p):
    d = xd.shape[1]
    n2 = 2 * d
    parts = [q[:, :, :N] for q in parts]
    grid = N // RB

    def bodyA(*refs):
        _combA_body(refs[0], refs[1], refs[2], *refs[3:])

    h2, sums = pl.pallas_call(
        bodyA,
        grid=(grid,),
        in_specs=[
            pl.BlockSpec((RB, d), lambda i: (i, 0)),
            pl.BlockSpec((d, n2), lambda i: (0, 0)),
            pl.BlockSpec((1, n2), lambda i: (0, 0)),
        ] + [
            pl.BlockSpec((2, 2, RB, D), lambda i: (0, 0, i, 0))
            for _ in parts
        ],
        out_specs=[
            pl.BlockSpec((RB, n2), lambda i: (i, 0)),
            pl.BlockSpec((8, n2), lambda i: (0, 0)),
        ],
        out_shape=[
            jax.ShapeDtypeStruct((N, n2), jnp.float32),
            jax.ShapeDtypeStruct((8, n2), jnp.float32),
        ],
    )(xd, p['W1'], p['b1'].reshape(1, n2), *parts)
    return pl.pallas_call(
        _combB_body,
        grid=(grid,),
        in_specs=[
            pl.BlockSpec((RB, n2), lambda i: (i, 0)),
            pl.BlockSpec((8, n2), lambda i: (0, 0)),
            pl.BlockSpec((1, n2), lambda i: (0, 0)),
            pl.BlockSpec((1, n2), lambda i: (0, 0)),
            pl.BlockSpec((n2, d), lambda i: (0, 0)),
            pl.BlockSpec((1, d), lambda i: (0, 0)),
        ],
        out_specs=pl.BlockSpec((RB, d), lambda i: (i, 0)),
        out_shape=jax.ShapeDtypeStruct((N, d), jnp.float32),
    )(h2, sums, p['g'].reshape(1, n2), p['bt'].reshape(1, n2),
      p['W2'], p['b2'].reshape(1, d))


def _head_body(z_ref, batch_ref, w1_ref, b1_ref, w2_ref, b2_ref, o_ref):
    onehot = (batch_ref[...] == lax.broadcasted_iota(
        jnp.int32, (N, NUM_GRAPHS), 1)).astype(jnp.float32)
    ssum = lax.dot_general(onehot, z_ref[...], (((0,), (0,)), ((), ())),
                           preferred_element_type=jnp.float32)
    cnt = lax.dot_general(onehot, jnp.ones((N, 1), jnp.float32),
                          (((0,), (0,)), ((), ())),
                          preferred_element_type=jnp.float32)
    pooled = ssum / jnp.maximum(cnt, 1.0)
    h = (jnp.dot(pooled, w1_ref[...], preferred_element_type=jnp.float32)
         + b1_ref[...])
    h = (jnp.dot(h, w2_ref[...], preferred_element_type=jnp.float32)
         + b2_ref[...])
    mx = jnp.max(h, axis=1, keepdims=True)
    sh = h - mx
    lse = jnp.log(jnp.sum(jnp.exp(sh), axis=1, keepdims=True))
    o_ref[...] = sh - lse


def _head(z, batch, params):
    return pl.pallas_call(
        _head_body,
        out_shape=jax.ShapeDtypeStruct((NUM_GRAPHS, OUT_DIM), jnp.float32),
    )(z, batch.reshape(N, 1), params['d1W'],
      params['d1b'].reshape(1, 64), params['d2W'],
      params['d2b'].reshape(1, OUT_DIM))


def _conv(p, x, src, dst, edge_attr):
    din = x.shape[1]
    if 'Wsrc' in p:
        dout = p['Wsrc'].shape[1]
        wcat = jnp.concatenate([p['Wsrc'], p['Wdst']], axis=1)
        bcat = jnp.concatenate([p['bsrc'], p['bdst']])
        hx = _linear(x, wcat, bcat)
        h, xd = hx[:, :dout], hx[:, dout:]
    else:
        dout = din
        h = x
        xd = x
    e = _linear(edge_attr, p['We'], p['be'], block_rows=4096)
    parts = [
        _edge_pass(h[:, k:k + D], e[:, k:k + D], src, dst)
        for k in range(0, dout, D)
    ]
    return _combine(parts, xd, p)


def kernel(x, edge_index, edge_attr, batch, params):
    src, dst = edge_index[0], edge_index[1]
    npad = E_PAD - E
    src = jnp.concatenate([src, jnp.zeros((npad,), jnp.int32)])
    dst = jnp.concatenate([dst, jnp.full((npad,), JUNK_ROW, jnp.int32)])
    edge_attr = jnp.pad(edge_attr, ((0, npad), (0, 0)))
    h = _conv(params['conv1'], x, src, dst, edge_attr)
    h = _conv(params['conv2'], h, src, dst, edge_attr)
    h = _conv(params['conv3'], h, src, dst, edge_attr)
    return _head(h, batch, params)


# speedup vs baseline: 1.0980x; 1.0980x over previous
import functools

import jax
import jax.numpy as jnp
from jax import lax
from jax.experimental import pallas as pl
from jax.experimental.pallas import tpu as pltpu
from jax.experimental.pallas import tpu_sc as plsc

N = 10000
E = 320000
NUM_GRAPHS = 64
OUT_DIM = 10
EPS = 1e-7

NC = 2
NS = 16
D = 64
EPB = 80
E_PER_SUB = 10240
E_PAD = NC * NS * E_PER_SUB
E_PER_CORE = E_PAD // NC
BLOCKS = E_PER_SUB // EPB
N_PAD = 10112
ROWS_PER_SUB = N_PAD // NS
ZROWS = 32
JUNK_ROW = N_PAD - 1


def _edge_body(h_hbm, e_hbm, src_hbm, dst_hbm, out_hbm,
               srcb, dstb, dstsc, hrows, erows, wbuf, mwbuf, zbuf,
               acc, s_src, s_dst, s_h, s_e, s_sw, s_sm):
    c = lax.axis_index("c")
    s = lax.axis_index("s")
    base = c * E_PER_CORE + s * E_PER_SUB

    def idx_start(j, b):
        off = base + j * EPB
        pltpu.async_copy(src_hbm.at[pl.ds(off, EPB)], srcb.at[b], s_src.at[b])
        pltpu.async_copy(dst_hbm.at[pl.ds(off, EPB)], dstb.at[b], s_dst.at[b])

    def idx_wait(b):
        pltpu.make_async_copy(src_hbm.at[pl.ds(0, EPB)], srcb.at[b],
                              s_src.at[b]).wait()
        pltpu.make_async_copy(dst_hbm.at[pl.ds(0, EPB)], dstb.at[b],
                              s_dst.at[b]).wait()

    def gat_start(j, b):
        off = base + j * EPB
        pltpu.async_copy(h_hbm.at[srcb.at[b]], hrows.at[b], s_h.at[b])
        pltpu.async_copy(e_hbm.at[pl.ds(off, EPB)], erows.at[b], s_e.at[b])

    def gat_wait(b):
        pltpu.make_async_copy(h_hbm.at[srcb.at[b]], hrows.at[b],
                              s_h.at[b]).wait()
        pltpu.make_async_copy(e_hbm.at[pl.ds(0, EPB)], erows.at[b],
                              s_e.at[b]).wait()

    def snap(b):
        for g in range(EPB // 16):
            sl = pl.ds(g * 16, 16)
            dstsc[b, sl] = dstb[b, sl]

    def comp(b):
        @pl.loop(0, EPB)
        def _(r):
            for g in range(D // 16):
                sl = pl.ds(g * 16, 16)
                m = jnp.maximum(hrows[b, r, sl] + erows[b, r, sl], 0.0) + EPS
                w = jnp.exp(m)
                wbuf[b, r, sl] = w
                mwbuf[b, r, sl] = m * w

    def sca_start(b):
        pltpu.async_copy(wbuf.at[b], acc.at[0].at[dstsc.at[b]], s_sw.at[b],
                         add=True)
        pltpu.async_copy(mwbuf.at[b], acc.at[1].at[dstsc.at[b]], s_sm.at[b],
                         add=True)

    def sca_wait(b):
        pltpu.make_async_copy(wbuf.at[b], acc.at[0].at[dstsc.at[b]],
                              s_sw.at[b]).wait()
        pltpu.make_async_copy(mwbuf.at[b], acc.at[1].at[dstsc.at[b]],
                              s_sm.at[b]).wait()

    zv = jnp.zeros((16,), jnp.float32)

    @pl.loop(0, ZROWS)
    def _(r):
        for g in range(D // 16):
            zbuf[r, pl.ds(g * 16, 16)] = zv

    for a in range(2):
        for k in range(ROWS_PER_SUB // ZROWS):
            pltpu.sync_copy(
                zbuf, acc.at[a, pl.ds(s * ROWS_PER_SUB + k * ZROWS, ZROWS)])
        rem = ROWS_PER_SUB % ZROWS
        if rem:
            pltpu.sync_copy(
                zbuf.at[pl.ds(0, rem)],
                acc.at[a, pl.ds(s * ROWS_PER_SUB
                                + (ROWS_PER_SUB // ZROWS) * ZROWS, rem)])
    plsc.subcore_barrier()

    idx_start(0, 0)
    idx_wait(0)
    gat_start(0, 0)
    idx_start(1, 1)
    gat_wait(0)
    idx_wait(1)
    gat_start(1, 1)
    snap(0)
    comp(0)
    sca_start(0)
    idx_start(2, 0)
    gat_wait(1)
    snap(1)
    idx_start(3, 1)
    comp(1)
    sca_start(1)
    idx_wait(0)
    gat_start(2, 0)

    @pl.loop(1, BLOCKS // 2 - 1)
    def _(g):
        j0 = 2 * g
        gat_wait(0)
        idx_wait(1)
        gat_start(j0 + 1, 1)
        sca_wait(0)
        snap(0)
        comp(0)
        sca_start(0)
        idx_start(j0 + 2, 0)
        gat_wait(1)
        sca_wait(1)
        snap(1)
        idx_start(j0 + 3, 1)
        comp(1)
        sca_start(1)
        idx_wait(0)
        gat_start(j0 + 2, 0)

    gat_wait(0)
    idx_wait(1)
    gat_start(BLOCKS - 1, 1)
    sca_wait(0)
    snap(0)
    comp(0)
    sca_start(0)
    gat_wait(1)
    sca_wait(1)
    snap(1)
    comp(1)
    sca_start(1)
    sca_wait(0)
    sca_wait(1)

    plsc.subcore_barrier()
    for a in range(2):
        pltpu.sync_copy(
            acc.at[a, pl.ds(s * ROWS_PER_SUB, ROWS_PER_SUB)],
            out_hbm.at[c, a, pl.ds(s * ROWS_PER_SUB, ROWS_PER_SUB)])


@jax.jit
def _edge_pass(h, e, src, dst):
    mesh = plsc.VectorSubcoreMesh(core_axis_name="c", subcore_axis_name="s")
    f = pl.kernel(
        _edge_body,
        out_type=jax.ShapeDtypeStruct((NC, 2, N_PAD, D), jnp.float32),
        mesh=mesh,
        scratch_types=[
            pltpu.VMEM((2, EPB), jnp.int32),
            pltpu.VMEM((2, EPB), jnp.int32),
            pltpu.VMEM((2, EPB), jnp.int32),
            pltpu.VMEM((2, EPB, D), jnp.float32),
            pltpu.VMEM((2, EPB, D), jnp.float32),
            pltpu.VMEM((2, EPB, D), jnp.float32),
            pltpu.VMEM((2, EPB, D), jnp.float32),
            pltpu.VMEM((ZROWS, D), jnp.float32),
            pltpu.VMEM_SHARED((2, N_PAD, D), jnp.float32),
            pltpu.SemaphoreType.DMA((2,)),
            pltpu.SemaphoreType.DMA((2,)),
            pltpu.SemaphoreType.DMA((2,)),
            pltpu.SemaphoreType.DMA((2,)),
            pltpu.SemaphoreType.DMA((2,)),
            pltpu.SemaphoreType.DMA((2,)),
        ],
        compiler_params=pltpu.CompilerParams(use_tc_tiling_on_sc=False),
    )
    return f(h, e, src, dst)


def _lin_body(a_ref, w_ref, b_ref, o_ref):
    o_ref[...] = (
        jnp.dot(a_ref[...], w_ref[...], preferred_element_type=jnp.float32)
        + b_ref[...])


def _linear(a, w, b, block_rows=None):
    m, k = a.shape
    n = w.shape[1]
    if block_rows is None:
        return pl.pallas_call(
            _lin_body,
            out_shape=jax.ShapeDtypeStruct((m, n), jnp.float32),
        )(a, w, b.reshape(1, n))
    grid = m // block_rows
    return pl.pallas_call(
        _lin_body,
        grid=(grid,),
        in_specs=[
            pl.BlockSpec((block_rows, k), lambda i: (i, 0)),
            pl.BlockSpec((k, n), lambda i: (0, 0)),
            pl.BlockSpec((1, n), lambda i: (0, 0)),
        ],
        out_specs=pl.BlockSpec((block_rows, n), lambda i: (i, 0)),
        out_shape=jax.ShapeDtypeStruct((m, n), jnp.float32),
    )(a, w, b.reshape(1, n))


RB = 2000


def _combA_body(xd_ref, w1_ref, b1_ref, *rest):
    p_refs = rest[:-2]
    h2_ref, sums_ref = rest[-2:]
    aggs = []
    for pr in p_refs:
        w = pr[0, 0, :, :] + pr[1, 0, :, :]
        mw = pr[0, 1, :, :] + pr[1, 1, :, :]
        aggs.append(mw / (w + 1e-16))
    agg = aggs[0] if len(aggs) == 1 else jnp.concatenate(aggs, axis=1)
    out = agg + xd_ref[...]
    h2 = (jnp.dot(out, w1_ref[...], preferred_element_type=jnp.float32)
          + b1_ref[...])
    h2_ref[...] = h2
    s1 = jnp.sum(h2, axis=0, keepdims=True)
    s2 = jnp.sum(h2 * h2, axis=0, keepdims=True)

    @pl.when(pl.program_id(0) == 0)
    def _():
        sums_ref[...] = jnp.zeros_like(sums_ref)

    sums_ref[0:1, :] += s1
    sums_ref[1:2, :] += s2


def _combB_body(h2_ref, sums_ref, g_ref, bt_ref, w2_ref, b2_ref, o_ref):
    mu = sums_ref[0:1, :] / N
    var = sums_ref[1:2, :] / N - mu * mu
    h2 = (h2_ref[...] - mu) * lax.rsqrt(var + 1e-5) * g_ref[...] + bt_ref[...]
    h2 = jnp.maximum(h2, 0.0)
    z = (jnp.dot(h2, w2_ref[...], preferred_element_type=jnp.float32)
         + b2_ref[...])
    o_ref[...] = jnp.maximum(z, 0.0)


def _combine(parts, xd, p):
    d = xd.shape[1]
    n2 = 2 * d
    parts = [q[:, :, :N] for q in parts]
    grid = N // RB

    def bodyA(*refs):
        _combA_body(refs[0], refs[1], refs[2], *refs[3:])

    h2, sums = pl.pallas_call(
        bodyA,
        grid=(grid,),
        in_specs=[
            pl.BlockSpec((RB, d), lambda i: (i, 0)),
            pl.BlockSpec((d, n2), lambda i: (0, 0)),
            pl.BlockSpec((1, n2), lambda i: (0, 0)),
        ] + [
            pl.BlockSpec((2, 2, RB, D), lambda i: (0, 0, i, 0))
            for _ in parts
        ],
        out_specs=[
            pl.BlockSpec((RB, n2), lambda i: (i, 0)),
            pl.BlockSpec((8, n2), lambda i: (0, 0)),
        ],
        out_shape=[
            jax.ShapeDtypeStruct((N, n2), jnp.float32),
            jax.ShapeDtypeStruct((8, n2), jnp.float32),
        ],
    )(xd, p['W1'], p['b1'].reshape(1, n2), *parts)
    return pl.pallas_call(
        _combB_body,
        grid=(grid,),
        in_specs=[
            pl.BlockSpec((RB, n2), lambda i: (i, 0)),
            pl.BlockSpec((8, n2), lambda i: (0, 0)),
            pl.BlockSpec((1, n2), lambda i: (0, 0)),
            pl.BlockSpec((1, n2), lambda i: (0, 0)),
            pl.BlockSpec((n2, d), lambda i: (0, 0)),
            pl.BlockSpec((1, d), lambda i: (0, 0)),
        ],
        out_specs=pl.BlockSpec((RB, d), lambda i: (i, 0)),
        out_shape=jax.ShapeDtypeStruct((N, d), jnp.float32),
    )(h2, sums, p['g'].reshape(1, n2), p['bt'].reshape(1, n2),
      p['W2'], p['b2'].reshape(1, d))


def _head_body(z_ref, batch_ref, w1_ref, b1_ref, w2_ref, b2_ref, o_ref):
    onehot = (batch_ref[...] == lax.broadcasted_iota(
        jnp.int32, (N, NUM_GRAPHS), 1)).astype(jnp.float32)
    ssum = lax.dot_general(onehot, z_ref[...], (((0,), (0,)), ((), ())),
                           preferred_element_type=jnp.float32)
    cnt = lax.dot_general(onehot, jnp.ones((N, 1), jnp.float32),
                          (((0,), (0,)), ((), ())),
                          preferred_element_type=jnp.float32)
    pooled = ssum / jnp.maximum(cnt, 1.0)
    h = (jnp.dot(pooled, w1_ref[...], preferred_element_type=jnp.float32)
         + b1_ref[...])
    h = (jnp.dot(h, w2_ref[...], preferred_element_type=jnp.float32)
         + b2_ref[...])
    mx = jnp.max(h, axis=1, keepdims=True)
    sh = h - mx
    lse = jnp.log(jnp.sum(jnp.exp(sh), axis=1, keepdims=True))
    o_ref[...] = sh - lse


def _head(z, batch, params):
    return pl.pallas_call(
        _head_body,
        out_shape=jax.ShapeDtypeStruct((NUM_GRAPHS, OUT_DIM), jnp.float32),
    )(z, batch.reshape(N, 1), params['d1W'],
      params['d1b'].reshape(1, 64), params['d2W'],
      params['d2b'].reshape(1, OUT_DIM))


def _conv(p, x, src, dst, edge_attr):
    din = x.shape[1]
    if 'Wsrc' in p:
        dout = p['Wsrc'].shape[1]
        wcat = jnp.concatenate([p['Wsrc'], p['Wdst']], axis=1)
        bcat = jnp.concatenate([p['bsrc'], p['bdst']])
        hx = _linear(x, wcat, bcat)
        h, xd = hx[:, :dout], hx[:, dout:]
    else:
        dout = din
        h = x
        xd = x
    e = _linear(edge_attr, p['We'], p['be'], block_rows=4096)
    parts = [
        _edge_pass(h[:, k:k + D], e[:, k:k + D], src, dst)
        for k in range(0, dout, D)
    ]
    return _combine(parts, xd, p)


def kernel(x, edge_index, edge_attr, batch, params):
    src, dst = edge_index[0], edge_index[1]
    npad = E_PAD - E
    src = jnp.concatenate([src, jnp.zeros((npad,), jnp.int32)])
    dst = jnp.concatenate([dst, jnp.full((npad,), JUNK_ROW, jnp.int32)])
    edge_attr = jnp.pad(edge_attr, ((0, npad), (0, 0)))
    h = _conv(params['conv1'], x, src, dst, edge_attr)
    h = _conv(params['conv2'], h, src, dst, edge_attr)
    h = _conv(params['conv3'], h, src, dst, edge_attr)
    return _head(h, batch, params)


# speedup vs baseline: 1.1361x; 1.0348x over previous
import functools

import jax
import jax.numpy as jnp
from jax import lax
from jax.experimental import pallas as pl
from jax.experimental.pallas import tpu as pltpu
from jax.experimental.pallas import tpu_sc as plsc

N = 10000
E = 320000
NUM_GRAPHS = 64
OUT_DIM = 10
EPS = 1e-7

NC = 2
NS = 16
D = 64
EPB = 80
E_PER_SUB = 10240
E_PAD = NC * NS * E_PER_SUB
E_PER_CORE = E_PAD // NC
BLOCKS = E_PER_SUB // EPB
N_PAD = 10112
ROWS_PER_SUB = N_PAD // NS
ZROWS = 32
JUNK_ROW = N_PAD - 1


E_PER_SUB3 = E_PAD // NS
BLOCKS3 = E_PER_SUB3 // EPB


def _pipeline(h_hbm, e_hbm, src_hbm, dst_hbm, base, nblocks,
              srcb, dstb, dstsc, hrows, erows, wbuf, mwbuf, acc,
              s_src, s_dst, s_h, s_e, s_sw, s_sm):
    def idx_start(j, b):
        off = base + j * EPB
        pltpu.async_copy(src_hbm.at[pl.ds(off, EPB)], srcb.at[b], s_src.at[b])
        pltpu.async_copy(dst_hbm.at[pl.ds(off, EPB)], dstb.at[b], s_dst.at[b])

    def idx_wait(b):
        pltpu.make_async_copy(src_hbm.at[pl.ds(0, EPB)], srcb.at[b],
                              s_src.at[b]).wait()
        pltpu.make_async_copy(dst_hbm.at[pl.ds(0, EPB)], dstb.at[b],
                              s_dst.at[b]).wait()

    def gat_start(j, b):
        off = base + j * EPB
        pltpu.async_copy(h_hbm.at[srcb.at[b]], hrows.at[b], s_h.at[b])
        pltpu.async_copy(e_hbm.at[pl.ds(off, EPB)], erows.at[b], s_e.at[b])

    def gat_wait(b):
        pltpu.make_async_copy(h_hbm.at[srcb.at[b]], hrows.at[b],
                              s_h.at[b]).wait()
        pltpu.make_async_copy(e_hbm.at[pl.ds(0, EPB)], erows.at[b],
                              s_e.at[b]).wait()

    def snap(b):
        for g in range(EPB // 16):
            sl = pl.ds(g * 16, 16)
            dstsc[b, sl] = dstb[b, sl]

    def comp(b):
        @pl.loop(0, EPB)
        def _(r):
            for g in range(D // 16):
                sl = pl.ds(g * 16, 16)
                m = jnp.maximum(hrows[b, r, sl] + erows[b, r, sl], 0.0) + EPS
                w = jnp.exp(m)
                wbuf[b, r, sl] = w
                mwbuf[b, r, sl] = m * w

    def sca_start(b):
        pltpu.async_copy(wbuf.at[b], acc.at[0].at[dstsc.at[b]], s_sw.at[b],
                         add=True)
        pltpu.async_copy(mwbuf.at[b], acc.at[1].at[dstsc.at[b]], s_sm.at[b],
                         add=True)

    def sca_wait(b):
        pltpu.make_async_copy(wbuf.at[b], acc.at[0].at[dstsc.at[b]],
                              s_sw.at[b]).wait()
        pltpu.make_async_copy(mwbuf.at[b], acc.at[1].at[dstsc.at[b]],
                              s_sm.at[b]).wait()

    idx_start(0, 0)
    idx_wait(0)
    gat_start(0, 0)
    idx_start(1, 1)
    gat_wait(0)
    idx_wait(1)
    gat_start(1, 1)
    snap(0)
    comp(0)
    sca_start(0)
    idx_start(2, 0)
    gat_wait(1)
    snap(1)
    idx_start(3, 1)
    comp(1)
    sca_start(1)
    idx_wait(0)
    gat_start(2, 0)

    @pl.loop(1, nblocks // 2 - 1)
    def _(g):
        j0 = 2 * g
        gat_wait(0)
        idx_wait(1)
        gat_start(j0 + 1, 1)
        sca_wait(0)
        snap(0)
        comp(0)
        sca_start(0)
        idx_start(j0 + 2, 0)
        gat_wait(1)
        sca_wait(1)
        snap(1)
        idx_start(j0 + 3, 1)
        comp(1)
        sca_start(1)
        idx_wait(0)
        gat_start(j0 + 2, 0)

    gat_wait(0)
    idx_wait(1)
    gat_start(nblocks - 1, 1)
    sca_wait(0)
    snap(0)
    comp(0)
    sca_start(0)
    gat_wait(1)
    sca_wait(1)
    snap(1)
    comp(1)
    sca_start(1)
    sca_wait(0)
    sca_wait(1)


def _zero_acc(s, zbuf, acc):
    zv = jnp.zeros((16,), jnp.float32)

    @pl.loop(0, ZROWS)
    def _(r):
        for g in range(D // 16):
            zbuf[r, pl.ds(g * 16, 16)] = zv

    for a in range(2):
        for k in range(ROWS_PER_SUB // ZROWS):
            pltpu.sync_copy(
                zbuf, acc.at[a, pl.ds(s * ROWS_PER_SUB + k * ZROWS, ZROWS)])
        rem = ROWS_PER_SUB % ZROWS
        if rem:
            pltpu.sync_copy(
                zbuf.at[pl.ds(0, rem)],
                acc.at[a, pl.ds(s * ROWS_PER_SUB
                                + (ROWS_PER_SUB // ZROWS) * ZROWS, rem)])


def _flush_acc(c, s, acc, out_hbm):
    for a in range(2):
        pltpu.sync_copy(
            acc.at[a, pl.ds(s * ROWS_PER_SUB, ROWS_PER_SUB)],
            out_hbm.at[c, a, pl.ds(s * ROWS_PER_SUB, ROWS_PER_SUB)])


def _edge_body(h_hbm, e_hbm, src_hbm, dst_hbm, out_hbm,
               srcb, dstb, dstsc, hrows, erows, wbuf, mwbuf, zbuf,
               acc, *sems):
    c = lax.axis_index("c")
    s = lax.axis_index("s")
    _zero_acc(s, zbuf, acc)
    plsc.subcore_barrier()
    _pipeline(h_hbm, e_hbm, src_hbm, dst_hbm,
              c * E_PER_CORE + s * E_PER_SUB, BLOCKS,
              srcb, dstb, dstsc, hrows, erows, wbuf, mwbuf, acc, *sems)
    plsc.subcore_barrier()
    _flush_acc(c, s, acc, out_hbm)


def _edge_body3(ha_hbm, hb_hbm, ea_hbm, eb_hbm, src_hbm, dst_hbm, out_hbm,
                srcb, dstb, dstsc, hrows, erows, wbuf, mwbuf, zbuf,
                acc, *sems):
    c = lax.axis_index("c")
    s = lax.axis_index("s")
    _zero_acc(s, zbuf, acc)
    plsc.subcore_barrier()
    base = s * E_PER_SUB3

    @pl.when(c == 0)
    def _():
        _pipeline(ha_hbm, ea_hbm, src_hbm, dst_hbm, base, BLOCKS3,
                  srcb, dstb, dstsc, hrows, erows, wbuf, mwbuf, acc, *sems)

    @pl.when(c == 1)
    def _():
        _pipeline(hb_hbm, eb_hbm, src_hbm, dst_hbm, base, BLOCKS3,
                  srcb, dstb, dstsc, hrows, erows, wbuf, mwbuf, acc, *sems)

    plsc.subcore_barrier()
    _flush_acc(c, s, acc, out_hbm)


_SCRATCH = [
    pltpu.VMEM((2, EPB), jnp.int32),
    pltpu.VMEM((2, EPB), jnp.int32),
    pltpu.VMEM((2, EPB), jnp.int32),
    pltpu.VMEM((2, EPB, D), jnp.float32),
    pltpu.VMEM((2, EPB, D), jnp.float32),
    pltpu.VMEM((2, EPB, D), jnp.float32),
    pltpu.VMEM((2, EPB, D), jnp.float32),
    pltpu.VMEM((ZROWS, D), jnp.float32),
    pltpu.VMEM_SHARED((2, N_PAD, D), jnp.float32),
    pltpu.SemaphoreType.DMA((2,)),
    pltpu.SemaphoreType.DMA((2,)),
    pltpu.SemaphoreType.DMA((2,)),
    pltpu.SemaphoreType.DMA((2,)),
    pltpu.SemaphoreType.DMA((2,)),
    pltpu.SemaphoreType.DMA((2,)),
]


@jax.jit
def _edge_pass3(ha, hb, ea, eb, src, dst):
    mesh = plsc.VectorSubcoreMesh(core_axis_name="c", subcore_axis_name="s")
    f = pl.kernel(
        _edge_body3,
        out_type=jax.ShapeDtypeStruct((NC, 2, N_PAD, D), jnp.float32),
        mesh=mesh,
        scratch_types=list(_SCRATCH),
        compiler_params=pltpu.CompilerParams(use_tc_tiling_on_sc=False),
    )
    return f(ha, hb, ea, eb, src, dst)


@jax.jit
def _edge_pass(h, e, src, dst):
    mesh = plsc.VectorSubcoreMesh(core_axis_name="c", subcore_axis_name="s")
    f = pl.kernel(
        _edge_body,
        out_type=jax.ShapeDtypeStruct((NC, 2, N_PAD, D), jnp.float32),
        mesh=mesh,
        scratch_types=list(_SCRATCH),
        compiler_params=pltpu.CompilerParams(use_tc_tiling_on_sc=False),
    )
    return f(h, e, src, dst)


def _lin_body(a_ref, w_ref, b_ref, o_ref):
    o_ref[...] = (
        jnp.dot(a_ref[...], w_ref[...], preferred_element_type=jnp.float32)
        + b_ref[...])


def _linear(a, w, b, block_rows=None):
    m, k = a.shape
    n = w.shape[1]
    if block_rows is None:
        return pl.pallas_call(
            _lin_body,
            out_shape=jax.ShapeDtypeStruct((m, n), jnp.float32),
        )(a, w, b.reshape(1, n))
    grid = m // block_rows
    return pl.pallas_call(
        _lin_body,
        grid=(grid,),
        in_specs=[
            pl.BlockSpec((block_rows, k), lambda i: (i, 0)),
            pl.BlockSpec((k, n), lambda i: (0, 0)),
            pl.BlockSpec((1, n), lambda i: (0, 0)),
        ],
        out_specs=pl.BlockSpec((block_rows, n), lambda i: (i, 0)),
        out_shape=jax.ShapeDtypeStruct((m, n), jnp.float32),
    )(a, w, b.reshape(1, n))


RB = 2000


def _combA_body(xd_ref, w1_ref, b1_ref, *rest):
    p_refs = rest[:-2]
    h2_ref, sums_ref = rest[-2:]
    aggs = []
    for pr in p_refs:
        if pr.shape[0] == 2 and len(pr.shape) == 4:
            w = pr[0, 0, :, :] + pr[1, 0, :, :]
            mw = pr[0, 1, :, :] + pr[1, 1, :, :]
        else:
            w = pr[0, :, :]
            mw = pr[1, :, :]
        aggs.append(mw / (w + 1e-16))
    agg = aggs[0] if len(aggs) == 1 else jnp.concatenate(aggs, axis=1)
    out = agg + xd_ref[...]
    h2 = (jnp.dot(out, w1_ref[...], preferred_element_type=jnp.float32)
          + b1_ref[...])
    h2_ref[...] = h2
    s1 = jnp.sum(h2, axis=0, keepdims=True)
    s2 = jnp.sum(h2 * h2, axis=0, keepdims=True)

    @pl.when(pl.program_id(0) == 0)
    def _():
        sums_ref[...] = jnp.zeros_like(sums_ref)

    sums_ref[0:1, :] += s1
    sums_ref[1:2, :] += s2


def _combB_body(h2_ref, sums_ref, g_ref, bt_ref, w2_ref, b2_ref, o_ref):
    mu = sums_ref[0:1, :] / N
    var = sums_ref[1:2, :] / N - mu * mu
    h2 = (h2_ref[...] - mu) * lax.rsqrt(var + 1e-5) * g_ref[...] + bt_ref[...]
    h2 = jnp.maximum(h2, 0.0)
    z = (jnp.dot(h2, w2_ref[...], preferred_element_type=jnp.float32)
         + b2_ref[...])
    o_ref[...] = jnp.maximum(z, 0.0)


def _combine(parts, xd, p):
    d = xd.shape[1]
    n2 = 2 * d
    parts = [q[:, :, :N] if q.ndim == 4 else q[:, :N] for q in parts]
    grid = N // RB

    def bodyA(*refs):
        _combA_body(refs[0], refs[1], refs[2], *refs[3:])

    h2, sums = pl.pallas_call(
        bodyA,
        grid=(grid,),
        in_specs=[
            pl.BlockSpec((RB, d), lambda i: (i, 0)),
            pl.BlockSpec((d, n2), lambda i: (0, 0)),
            pl.BlockSpec((1, n2), lambda i: (0, 0)),
        ] + [
            pl.BlockSpec((2, 2, RB, D), lambda i: (0, 0, i, 0))
            if q.ndim == 4 else pl.BlockSpec((2, RB, D), lambda i: (0, i, 0))
            for q in parts
        ],
        out_specs=[
            pl.BlockSpec((RB, n2), lambda i: (i, 0)),
            pl.BlockSpec((8, n2), lambda i: (0, 0)),
        ],
        out_shape=[
            jax.ShapeDtypeStruct((N, n2), jnp.float32),
            jax.ShapeDtypeStruct((8, n2), jnp.float32),
        ],
    )(xd, p['W1'], p['b1'].reshape(1, n2), *parts)
    return pl.pallas_call(
        _combB_body,
        grid=(grid,),
        in_specs=[
            pl.BlockSpec((RB, n2), lambda i: (i, 0)),
            pl.BlockSpec((8, n2), lambda i: (0, 0)),
            pl.BlockSpec((1, n2), lambda i: (0, 0)),
            pl.BlockSpec((1, n2), lambda i: (0, 0)),
            pl.BlockSpec((n2, d), lambda i: (0, 0)),
            pl.BlockSpec((1, d), lambda i: (0, 0)),
        ],
        out_specs=pl.BlockSpec((RB, d), lambda i: (i, 0)),
        out_shape=jax.ShapeDtypeStruct((N, d), jnp.float32),
    )(h2, sums, p['g'].reshape(1, n2), p['bt'].reshape(1, n2),
      p['W2'], p['b2'].reshape(1, d))


def _head_body(z_ref, batch_ref, w1_ref, b1_ref, w2_ref, b2_ref, o_ref):
    onehot = (batch_ref[...] == lax.broadcasted_iota(
        jnp.int32, (N, NUM_GRAPHS), 1)).astype(jnp.float32)
    ssum = lax.dot_general(onehot, z_ref[...], (((0,), (0,)), ((), ())),
                           preferred_element_type=jnp.float32)
    cnt = lax.dot_general(onehot, jnp.ones((N, 1), jnp.float32),
                          (((0,), (0,)), ((), ())),
                          preferred_element_type=jnp.float32)
    pooled = ssum / jnp.maximum(cnt, 1.0)
    h = (jnp.dot(pooled, w1_ref[...], preferred_element_type=jnp.float32)
         + b1_ref[...])
    h = (jnp.dot(h, w2_ref[...], preferred_element_type=jnp.float32)
         + b2_ref[...])
    mx = jnp.max(h, axis=1, keepdims=True)
    sh = h - mx
    lse = jnp.log(jnp.sum(jnp.exp(sh), axis=1, keepdims=True))
    o_ref[...] = sh - lse


def _head(z, batch, params):
    return pl.pallas_call(
        _head_body,
        out_shape=jax.ShapeDtypeStruct((NUM_GRAPHS, OUT_DIM), jnp.float32),
    )(z, batch.reshape(N, 1), params['d1W'],
      params['d1b'].reshape(1, 64), params['d2W'],
      params['d2b'].reshape(1, OUT_DIM))


def _conv(p, x, src, dst, edge_attr):
    din = x.shape[1]
    if 'Wsrc' in p:
        dout = p['Wsrc'].shape[1]
        wcat = jnp.concatenate([p['Wsrc'], p['Wdst']], axis=1)
        bcat = jnp.concatenate([p['bsrc'], p['bdst']])
        hx = _linear(x, wcat, bcat)
        h, xd = hx[:, :dout], hx[:, dout:]
    else:
        dout = din
        h = x
        xd = x
    e = _linear(edge_attr, p['We'], p['be'], block_rows=4096)
    if dout == D:
        parts = [_edge_pass(h, e, src, dst)]
    else:
        p3 = _edge_pass3(h[:, :D], h[:, D:], e[:, :D], e[:, D:], src, dst)
        parts = [p3[0], p3[1]]
    return _combine(parts, xd, p)


def kernel(x, edge_index, edge_attr, batch, params):
    src, dst = edge_index[0], edge_index[1]
    npad = E_PAD - E
    src = jnp.concatenate([src, jnp.zeros((npad,), jnp.int32)])
    dst = jnp.concatenate([dst, jnp.full((npad,), JUNK_ROW, jnp.int32)])
    edge_attr = jnp.pad(edge_attr, ((0, npad), (0, 0)))
    h = _conv(params['conv1'], x, src, dst, edge_attr)
    h = _conv(params['conv2'], h, src, dst, edge_attr)
    h = _conv(params['conv3'], h, src, dst, edge_attr)
    return _head(h, batch, params)


# speedup vs baseline: 1.1368x; 1.0006x over previous
import functools

import jax
import jax.numpy as jnp
from jax import lax
from jax.experimental import pallas as pl
from jax.experimental.pallas import tpu as pltpu
from jax.experimental.pallas import tpu_sc as plsc

N = 10000
E = 320000
NUM_GRAPHS = 64
OUT_DIM = 10
EPS = 1e-7

NC = 2
NS = 16
D = 64
EPB = 80
E_PER_SUB = 10240
E_PAD = NC * NS * E_PER_SUB
E_PER_CORE = E_PAD // NC
BLOCKS = E_PER_SUB // EPB
N_PAD = 10112
ROWS_PER_SUB = N_PAD // NS
ZROWS = 32
JUNK_ROW = N_PAD - 1


E_PER_SUB3 = E_PAD // NS
BLOCKS3 = E_PER_SUB3 // EPB


def _pipeline(h_hbm, e_hbm, src_hbm, dst_hbm, base, nblocks,
              srcb, dstb, dstsc, hrows, erows, wbuf, mwbuf, acc,
              s_src, s_dst, s_h, s_e, s_sw, s_sm):
    def idx_start(j, b):
        off = base + j * EPB
        pltpu.async_copy(src_hbm.at[pl.ds(off, EPB)], srcb.at[b], s_src.at[b])
        pltpu.async_copy(dst_hbm.at[pl.ds(off, EPB)], dstb.at[b], s_dst.at[b])

    def idx_wait(b):
        pltpu.make_async_copy(src_hbm.at[pl.ds(0, EPB)], srcb.at[b],
                              s_src.at[b]).wait()
        pltpu.make_async_copy(dst_hbm.at[pl.ds(0, EPB)], dstb.at[b],
                              s_dst.at[b]).wait()

    def gat_start(j, b):
        off = base + j * EPB
        pltpu.async_copy(h_hbm.at[srcb.at[b]], hrows.at[b], s_h.at[b])
        pltpu.async_copy(e_hbm.at[pl.ds(off, EPB)], erows.at[b], s_e.at[b])

    def gat_wait(b):
        pltpu.make_async_copy(h_hbm.at[srcb.at[b]], hrows.at[b],
                              s_h.at[b]).wait()
        pltpu.make_async_copy(e_hbm.at[pl.ds(0, EPB)], erows.at[b],
                              s_e.at[b]).wait()

    def snap(b):
        for g in range(EPB // 16):
            sl = pl.ds(g * 16, 16)
            dstsc[b, sl] = dstb[b, sl]

    def comp(b):
        @pl.loop(0, EPB)
        def _(r):
            for g in range(D // 16):
                sl = pl.ds(g * 16, 16)
                m = jnp.maximum(hrows[b, r, sl] + erows[b, r, sl], 0.0) + EPS
                w = jnp.exp(m)
                wbuf[b, r, sl] = w
                mwbuf[b, r, sl] = m * w

    def sca_start(b):
        pltpu.async_copy(wbuf.at[b], acc.at[0].at[dstsc.at[b]], s_sw.at[b],
                         add=True)
        pltpu.async_copy(mwbuf.at[b], acc.at[1].at[dstsc.at[b]], s_sm.at[b],
                         add=True)

    def sca_wait(b):
        pltpu.make_async_copy(wbuf.at[b], acc.at[0].at[dstsc.at[b]],
                              s_sw.at[b]).wait()
        pltpu.make_async_copy(mwbuf.at[b], acc.at[1].at[dstsc.at[b]],
                              s_sm.at[b]).wait()

    idx_start(0, 0)
    idx_wait(0)
    gat_start(0, 0)
    idx_start(1, 1)
    gat_wait(0)
    idx_wait(1)
    gat_start(1, 1)
    snap(0)
    comp(0)
    sca_start(0)
    idx_start(2, 0)
    gat_wait(1)
    snap(1)
    idx_start(3, 1)
    comp(1)
    sca_start(1)
    idx_wait(0)
    gat_start(2, 0)

    @pl.loop(1, nblocks // 2 - 1)
    def _(g):
        j0 = 2 * g
        gat_wait(0)
        idx_wait(1)
        gat_start(j0 + 1, 1)
        sca_wait(0)
        snap(0)
        comp(0)
        sca_start(0)
        idx_start(j0 + 2, 0)
        gat_wait(1)
        sca_wait(1)
        snap(1)
        idx_start(j0 + 3, 1)
        comp(1)
        sca_start(1)
        idx_wait(0)
        gat_start(j0 + 2, 0)

    gat_wait(0)
    idx_wait(1)
    gat_start(nblocks - 1, 1)
    sca_wait(0)
    snap(0)
    comp(0)
    sca_start(0)
    gat_wait(1)
    sca_wait(1)
    snap(1)
    comp(1)
    sca_start(1)
    sca_wait(0)
    sca_wait(1)


def _zero_acc(s, zbuf, acc):
    zv = jnp.zeros((16,), jnp.float32)

    @pl.loop(0, ZROWS)
    def _(r):
        for g in range(D // 16):
            zbuf[r, pl.ds(g * 16, 16)] = zv

    for a in range(2):
        for k in range(ROWS_PER_SUB // ZROWS):
            pltpu.sync_copy(
                zbuf, acc.at[a, pl.ds(s * ROWS_PER_SUB + k * ZROWS, ZROWS)])
        rem = ROWS_PER_SUB % ZROWS
        if rem:
            pltpu.sync_copy(
                zbuf.at[pl.ds(0, rem)],
                acc.at[a, pl.ds(s * ROWS_PER_SUB
                                + (ROWS_PER_SUB // ZROWS) * ZROWS, rem)])


def _flush_acc(c, s, acc, out_hbm):
    for a in range(2):
        pltpu.sync_copy(
            acc.at[a, pl.ds(s * ROWS_PER_SUB, ROWS_PER_SUB)],
            out_hbm.at[c, a, pl.ds(s * ROWS_PER_SUB, ROWS_PER_SUB)])


K0 = 112
K1 = 2 * BLOCKS - K0


def _edge_body(h_hbm, e_hbm, src_hbm, dst_hbm, out_hbm,
               srcb, dstb, dstsc, hrows, erows, wbuf, mwbuf, zbuf,
               acc, *sems):
    c = lax.axis_index("c")
    s = lax.axis_index("s")
    _zero_acc(s, zbuf, acc)
    plsc.subcore_barrier()

    @pl.when(c == 0)
    def _():
        _pipeline(h_hbm, e_hbm, src_hbm, dst_hbm, s * (K0 * EPB), K0,
                  srcb, dstb, dstsc, hrows, erows, wbuf, mwbuf, acc, *sems)

    @pl.when(c == 1)
    def _():
        _pipeline(h_hbm, e_hbm, src_hbm, dst_hbm,
                  NS * K0 * EPB + s * (K1 * EPB), K1,
                  srcb, dstb, dstsc, hrows, erows, wbuf, mwbuf, acc, *sems)

    plsc.subcore_barrier()
    _flush_acc(c, s, acc, out_hbm)


def _edge_body3(ha_hbm, hb_hbm, ea_hbm, eb_hbm, src_hbm, dst_hbm, out_hbm,
                srcb, dstb, dstsc, hrows, erows, wbuf, mwbuf, zbuf,
                acc, *sems):
    c = lax.axis_index("c")
    s = lax.axis_index("s")
    _zero_acc(s, zbuf, acc)
    plsc.subcore_barrier()
    base = s * E_PER_SUB3

    @pl.when(c == 0)
    def _():
        _pipeline(ha_hbm, ea_hbm, src_hbm, dst_hbm, base, BLOCKS3,
                  srcb, dstb, dstsc, hrows, erows, wbuf, mwbuf, acc, *sems)

    @pl.when(c == 1)
    def _():
        _pipeline(hb_hbm, eb_hbm, src_hbm, dst_hbm, base, BLOCKS3,
                  srcb, dstb, dstsc, hrows, erows, wbuf, mwbuf, acc, *sems)

    plsc.subcore_barrier()
    _flush_acc(c, s, acc, out_hbm)


_SCRATCH = [
    pltpu.VMEM((2, EPB), jnp.int32),
    pltpu.VMEM((2, EPB), jnp.int32),
    pltpu.VMEM((2, EPB), jnp.int32),
    pltpu.VMEM((2, EPB, D), jnp.float32),
    pltpu.VMEM((2, EPB, D), jnp.float32),
    pltpu.VMEM((2, EPB, D), jnp.float32),
    pltpu.VMEM((2, EPB, D), jnp.float32),
    pltpu.VMEM((ZROWS, D), jnp.float32),
    pltpu.VMEM_SHARED((2, N_PAD, D), jnp.float32),
    pltpu.SemaphoreType.DMA((2,)),
    pltpu.SemaphoreType.DMA((2,)),
    pltpu.SemaphoreType.DMA((2,)),
    pltpu.SemaphoreType.DMA((2,)),
    pltpu.SemaphoreType.DMA((2,)),
    pltpu.SemaphoreType.DMA((2,)),
]


@jax.jit
def _edge_pass3(ha, hb, ea, eb, src, dst):
    mesh = plsc.VectorSubcoreMesh(core_axis_name="c", subcore_axis_name="s")
    f = pl.kernel(
        _edge_body3,
        out_type=jax.ShapeDtypeStruct((NC, 2, N_PAD, D), jnp.float32),
        mesh=mesh,
        scratch_types=list(_SCRATCH),
        compiler_params=pltpu.CompilerParams(use_tc_tiling_on_sc=False),
    )
    return f(ha, hb, ea, eb, src, dst)


@jax.jit
def _edge_pass(h, e, src, dst):
    mesh = plsc.VectorSubcoreMesh(core_axis_name="c", subcore_axis_name="s")
    f = pl.kernel(
        _edge_body,
        out_type=jax.ShapeDtypeStruct((NC, 2, N_PAD, D), jnp.float32),
        mesh=mesh,
        scratch_types=list(_SCRATCH),
        compiler_params=pltpu.CompilerParams(use_tc_tiling_on_sc=False),
    )
    return f(h, e, src, dst)


def _lin_body(a_ref, w_ref, b_ref, o_ref):
    o_ref[...] = (
        jnp.dot(a_ref[...], w_ref[...], preferred_element_type=jnp.float32)
        + b_ref[...])


def _linear(a, w, b, block_rows=None):
    m, k = a.shape
    n = w.shape[1]
    if block_rows is None:
        return pl.pallas_call(
            _lin_body,
            out_shape=jax.ShapeDtypeStruct((m, n), jnp.float32),
        )(a, w, b.reshape(1, n))
    grid = m // block_rows
    return pl.pallas_call(
        _lin_body,
        grid=(grid,),
        in_specs=[
            pl.BlockSpec((block_rows, k), lambda i: (i, 0)),
            pl.BlockSpec((k, n), lambda i: (0, 0)),
            pl.BlockSpec((1, n), lambda i: (0, 0)),
        ],
        out_specs=pl.BlockSpec((block_rows, n), lambda i: (i, 0)),
        out_shape=jax.ShapeDtypeStruct((m, n), jnp.float32),
    )(a, w, b.reshape(1, n))


RB = 2000


def _combA_body(xd_ref, w1_ref, b1_ref, *rest):
    p_refs = rest[:-2]
    h2_ref, sums_ref = rest[-2:]
    aggs = []
    for pr in p_refs:
        if pr.shape[0] == 2 and len(pr.shape) == 4:
            w = pr[0, 0, :, :] + pr[1, 0, :, :]
            mw = pr[0, 1, :, :] + pr[1, 1, :, :]
        else:
            w = pr[0, :, :]
            mw = pr[1, :, :]
        aggs.append(mw / (w + 1e-16))
    agg = aggs[0] if len(aggs) == 1 else jnp.concatenate(aggs, axis=1)
    out = agg + xd_ref[...]
    h2 = (jnp.dot(out, w1_ref[...], preferred_element_type=jnp.float32)
          + b1_ref[...])
    h2_ref[...] = h2
    s1 = jnp.sum(h2, axis=0, keepdims=True)
    s2 = jnp.sum(h2 * h2, axis=0, keepdims=True)

    @pl.when(pl.program_id(0) == 0)
    def _():
        sums_ref[...] = jnp.zeros_like(sums_ref)

    sums_ref[0:1, :] += s1
    sums_ref[1:2, :] += s2


def _combB_body(h2_ref, sums_ref, g_ref, bt_ref, w2_ref, b2_ref, o_ref):
    mu = sums_ref[0:1, :] / N
    var = sums_ref[1:2, :] / N - mu * mu
    h2 = (h2_ref[...] - mu) * lax.rsqrt(var + 1e-5) * g_ref[...] + bt_ref[...]
    h2 = jnp.maximum(h2, 0.0)
    z = (jnp.dot(h2, w2_ref[...], preferred_element_type=jnp.float32)
         + b2_ref[...])
    o_ref[...] = jnp.maximum(z, 0.0)


def _combine(parts, xd, p):
    d = xd.shape[1]
    n2 = 2 * d
    parts = [q[:, :, :N] if q.ndim == 4 else q[:, :N] for q in parts]
    grid = N // RB

    def bodyA(*refs):
        _combA_body(refs[0], refs[1], refs[2], *refs[3:])

    h2, sums = pl.pallas_call(
        bodyA,
        grid=(grid,),
        in_specs=[
            pl.BlockSpec((RB, d), lambda i: (i, 0)),
            pl.BlockSpec((d, n2), lambda i: (0, 0)),
            pl.BlockSpec((1, n2), lambda i: (0, 0)),
        ] + [
            pl.BlockSpec((2, 2, RB, D), lambda i: (0, 0, i, 0))
            if q.ndim == 4 else pl.BlockSpec((2, RB, D), lambda i: (0, i, 0))
            for q in parts
        ],
        out_specs=[
            pl.BlockSpec((RB, n2), lambda i: (i, 0)),
            pl.BlockSpec((8, n2), lambda i: (0, 0)),
        ],
        out_shape=[
            jax.ShapeDtypeStruct((N, n2), jnp.float32),
            jax.ShapeDtypeStruct((8, n2), jnp.float32),
        ],
    )(xd, p['W1'], p['b1'].reshape(1, n2), *parts)
    return pl.pallas_call(
        _combB_body,
        grid=(grid,),
        in_specs=[
            pl.BlockSpec((RB, n2), lambda i: (i, 0)),
            pl.BlockSpec((8, n2), lambda i: (0, 0)),
            pl.BlockSpec((1, n2), lambda i: (0, 0)),
            pl.BlockSpec((1, n2), lambda i: (0, 0)),
            pl.BlockSpec((n2, d), lambda i: (0, 0)),
            pl.BlockSpec((1, d), lambda i: (0, 0)),
        ],
        out_specs=pl.BlockSpec((RB, d), lambda i: (i, 0)),
        out_shape=jax.ShapeDtypeStruct((N, d), jnp.float32),
    )(h2, sums, p['g'].reshape(1, n2), p['bt'].reshape(1, n2),
      p['W2'], p['b2'].reshape(1, d))


def _head_body(z_ref, batch_ref, w1_ref, b1_ref, w2_ref, b2_ref, o_ref):
    onehot = (batch_ref[...] == lax.broadcasted_iota(
        jnp.int32, (N, NUM_GRAPHS), 1)).astype(jnp.float32)
    ssum = lax.dot_general(onehot, z_ref[...], (((0,), (0,)), ((), ())),
                           preferred_element_type=jnp.float32)
    cnt = lax.dot_general(onehot, jnp.ones((N, 1), jnp.float32),
                          (((0,), (0,)), ((), ())),
                          preferred_element_type=jnp.float32)
    pooled = ssum / jnp.maximum(cnt, 1.0)
    h = (jnp.dot(pooled, w1_ref[...], preferred_element_type=jnp.float32)
         + b1_ref[...])
    h = (jnp.dot(h, w2_ref[...], preferred_element_type=jnp.float32)
         + b2_ref[...])
    mx = jnp.max(h, axis=1, keepdims=True)
    sh = h - mx
    lse = jnp.log(jnp.sum(jnp.exp(sh), axis=1, keepdims=True))
    o_ref[...] = sh - lse


def _head(z, batch, params):
    return pl.pallas_call(
        _head_body,
        out_shape=jax.ShapeDtypeStruct((NUM_GRAPHS, OUT_DIM), jnp.float32),
    )(z, batch.reshape(N, 1), params['d1W'],
      params['d1b'].reshape(1, 64), params['d2W'],
      params['d2b'].reshape(1, OUT_DIM))


def _conv(p, x, src, dst, edge_attr):
    din = x.shape[1]
    if 'Wsrc' in p:
        dout = p['Wsrc'].shape[1]
        wcat = jnp.concatenate([p['Wsrc'], p['Wdst']], axis=1)
        bcat = jnp.concatenate([p['bsrc'], p['bdst']])
        hx = _linear(x, wcat, bcat)
        h, xd = hx[:, :dout], hx[:, dout:]
    else:
        dout = din
        h = x
        xd = x
    e = _linear(edge_attr, p['We'], p['be'], block_rows=4096)
    if dout == D:
        parts = [_edge_pass(h, e, src, dst)]
    else:
        p3 = _edge_pass3(h[:, :D], h[:, D:], e[:, :D], e[:, D:], src, dst)
        parts = [p3[0], p3[1]]
    return _combine(parts, xd, p)


def kernel(x, edge_index, edge_attr, batch, params):
    src, dst = edge_index[0], edge_index[1]
    npad = E_PAD - E
    src = jnp.concatenate([src, jnp.zeros((npad,), jnp.int32)])
    dst = jnp.concatenate([dst, jnp.full((npad,), JUNK_ROW, jnp.int32)])
    edge_attr = jnp.pad(edge_attr, ((0, npad), (0, 0)))
    h = _conv(params['conv1'], x, src, dst, edge_attr)
    h = _conv(params['conv2'], h, src, dst, edge_attr)
    h = _conv(params['conv3'], h, src, dst, edge_attr)
    return _head(h, batch, params)
